# baseline (speedup 1.0000x reference)
import functools

import jax
import jax.numpy as jnp
from jax import lax
from jax.experimental import pallas as pl
from jax.experimental.pallas import tpu as pltpu
from jax.experimental.pallas import tpu_sc as plsc

B, S, E, V = 4, 2048, 768, 100000
NC, NS, L = 2, 16, 16
NW = NC * NS
SCHUNK = S // NW
EV = E // L
CH = 32
NCHUNK = (B * SCHUNK) // CH
NBUF = 3


def _make_kernel():
    mesh = plsc.VectorSubcoreMesh(core_axis_name="c", subcore_axis_name="s")

    @functools.partial(
        pl.kernel,
        out_type=jax.ShapeDtypeStruct((B, S, E), jnp.float32),
        mesh=mesh,
        compiler_params=pltpu.CompilerParams(use_tc_tiling_on_sc=False),
        scratch_types=[
            pltpu.VMEM((B, SCHUNK), jnp.int32),
            pltpu.VMEM((SCHUNK, E), jnp.float32),
            [pltpu.VMEM((CH, E), jnp.float32) for _ in range(NBUF)],
            [pltpu.SemaphoreType.DMA for _ in range(NBUF)],
            [pltpu.SemaphoreType.DMA for _ in range(NBUF)],
            pltpu.SemaphoreType.DMA,
        ],
    )
    def k(ids_hbm, tab_hbm, pos_hbm, out_hbm, idx_v, pos_v, bufs, gsems, wsems,
          psem):
        wid = lax.axis_index("s") * NC + lax.axis_index("c")
        s0 = wid * SCHUNK

        pos_cp = pltpu.async_copy(pos_hbm.at[pl.ds(s0, SCHUNK)], pos_v, psem)
        for b in range(B):
            pltpu.sync_copy(ids_hbm.at[b, pl.ds(s0, SCHUNK)], idx_v.at[b])

        def gather(c):
            b, h = c // 2, c % 2
            return pltpu.async_copy(
                tab_hbm.at[idx_v.at[b, pl.ds(h * CH, CH)]],
                bufs[c % NBUF],
                gsems[c % NBUF],
            )

        def write(c):
            b, h = c // 2, c % 2
            return pltpu.async_copy(
                bufs[c % NBUF],
                out_hbm.at[b, pl.ds(s0 + h * CH, CH)],
                wsems[c % NBUF],
            )

        g_cp = [None] * NCHUNK
        w_cp = [None] * NCHUNK
        g_cp[0] = gather(0)
        g_cp[1] = gather(1)
        pos_cp.wait()

        for c in range(NCHUNK):
            g_cp[c].wait()

            h = c % 2
            buf = bufs[c % NBUF]

            @plsc.parallel_loop(0, CH, 1, unroll=2)
            def add_row(r):
                for e in range(EV):
                    plsc.addupdate(
                        buf.at[r, pl.ds(e * L, L)],
                        pos_v[h * CH + r, pl.ds(e * L, L)],
                    )

            nc = c + 2
            if nc < NCHUNK:
                if c >= 1:
                    w_cp[c - 1].wait()
                g_cp[nc] = gather(nc)

            w_cp[c] = write(c)

        for c in (NCHUNK - 3, NCHUNK - 2, NCHUNK - 1):
            w_cp[c].wait()

    return k


_kernel = _make_kernel()


def kernel(input_ids, token_embeddings, position_embeddings):
    return _kernel(input_ids.astype(jnp.int32), token_embeddings,
                   position_embeddings)


# speedup vs baseline: 7.3006x; 7.3006x over previous
import functools

import jax
import jax.numpy as jnp
from jax import lax
from jax.experimental import pallas as pl
from jax.experimental.pallas import tpu as pltpu
from jax.experimental.pallas import tpu_sc as plsc

B, S, E, V = 4, 2048, 768, 100000
NC, NS, L = 2, 16, 16
NW = NC * NS
SCHUNK = S // NW
EV = E // L
CS = 8
NCHUNK = SCHUNK // CS
CH = B * CS
NBUF = 3


def _make_kernel():
    mesh = plsc.VectorSubcoreMesh(core_axis_name="c", subcore_axis_name="s")

    @functools.partial(
        pl.kernel,
        out_type=jax.ShapeDtypeStruct((B, S, E), jnp.float32),
        mesh=mesh,
        scratch_types=[
            pltpu.VMEM((NCHUNK, CH), jnp.int32),
            pltpu.VMEM((SCHUNK, E), jnp.float32),
            [pltpu.VMEM((CH, E), jnp.float32) for _ in range(NBUF)],
            [pltpu.SemaphoreType.DMA for _ in range(NBUF)],
            [pltpu.SemaphoreType.DMA for _ in range(NBUF)],
            pltpu.SemaphoreType.DMA,
            pltpu.SemaphoreType.DMA,
        ],
    )
    def k(ids_hbm, tab_hbm, pos_hbm, out_hbm, idx_v, pos_v, bufs, gsems, wsems,
          psem, isem):
        wid = lax.axis_index("s") * NC + lax.axis_index("c")
        s0 = wid * SCHUNK

        pos_cp = pltpu.async_copy(pos_hbm.at[pl.ds(s0, SCHUNK)], pos_v, psem)
        i_cp = [
            pltpu.async_copy(
                ids_hbm.at[b, pl.ds(s0 + c * CS, CS)],
                idx_v.at[c, pl.ds(b * CS, CS)],
                isem,
            )
            for c in range(NCHUNK)
            for b in range(B)
        ]
        for cp in i_cp:
            cp.wait()

        def gather(c):
            return pltpu.async_copy(
                tab_hbm.at[idx_v.at[c]],
                bufs[c % NBUF],
                gsems[c % NBUF],
            )

        def write(c):
            return [
                pltpu.async_copy(
                    bufs[c % NBUF].at[pl.ds(b * CS, CS)],
                    out_hbm.at[b, pl.ds(s0 + c * CS, CS)],
                    wsems[c % NBUF],
                )
                for b in range(B)
            ]

        g_cp = [None] * NCHUNK
        w_cp = [None] * NCHUNK
        g_cp[0] = gather(0)
        g_cp[1] = gather(1)
        pos_cp.wait()

        for c in range(NCHUNK):
            g_cp[c].wait()

            buf = bufs[c % NBUF]

            @plsc.parallel_loop(0, CS, 1)
            def add_row(sl):
                pr = c * CS + sl
                for e in range(EV):
                    pv = pos_v[pr, pl.ds(e * L, L)]
                    for b in range(B):
                        plsc.addupdate(
                            buf.at[b * CS + sl, pl.ds(e * L, L)], pv
                        )

            nc = c + 2
            if nc < NCHUNK:
                if c >= 1:
                    for w in w_cp[c - 1]:
                        w.wait()
                g_cp[nc] = gather(nc)

            w_cp[c] = write(c)

        for c in (NCHUNK - 3, NCHUNK - 2, NCHUNK - 1):
            for w in w_cp[c]:
                w.wait()

    return k


_kernel = _make_kernel()


def kernel(input_ids, token_embeddings, position_embeddings):
    return _kernel(input_ids.astype(jnp.int32), token_embeddings,
                   position_embeddings)
